# Initial kernel scaffold; baseline (speedup 1.0000x reference)
import jax
import jax.numpy as jnp
from jax.experimental import pallas as pl


def kernel(node_features, edge_features, Esrc, Etgt, batch, W_in, b_in, ee_W1, ee_b1, ee_W2, ee_b2, gru_Wih, gru_Whh, gru_bih, gru_bhh, lstm_Wih, lstm_Whh, lstm_bih, lstm_bhh, W_out, b_out):
    raise NotImplementedError("write your pallas kernel here")


# speedup vs baseline: 4.2660x; 4.2660x over previous
import functools

import jax
import jax.numpy as jnp
from jax import lax
from jax.experimental import pallas as pl
from jax.experimental.pallas import tpu as pltpu
from jax.experimental.pallas import tpu_sc as plsc

N = 10000
E = 160000
DF = 128
DE = 16
H = 32
OUTD = 1
NG = 16
T_MP = 3
T_S2S = 12

NC = 2
NS = 16
NW = NC * NS
CHUNK = 128
K_CH = 40
EPW = K_CH * CHUNK
E_PAD = NW * EPW
N_PAD = N + 8

BE = 1024
BN = 1000


def _sc_gather_body(h_hbm, idx_hbm, out_hbm, idxv, buf, sem):
    c = lax.axis_index("c")
    s = lax.axis_index("s")
    w = c * NS + s
    pltpu.sync_copy(idx_hbm.at[w], idxv)

    def body(j, carry):
        pltpu.async_copy(h_hbm.at[idxv.at[j]], buf, sem).wait()
        pltpu.sync_copy(buf, out_hbm.at[pl.ds(w * EPW + j * CHUNK, CHUNK)])
        return carry

    lax.fori_loop(0, K_CH, body, 0)


def _sc_scatter_body(msg_hbm, idx_hbm, zeros_hbm, out0_hbm, out1_hbm,
                     idxv, buf, acc, sem):
    c = lax.axis_index("c")
    s = lax.axis_index("s")
    w = c * NS + s

    @pl.when(s == 0)
    def _zero():
        pltpu.sync_copy(zeros_hbm, acc)

    plsc.subcore_barrier()
    pltpu.sync_copy(idx_hbm.at[w], idxv)

    def body(j, carry):
        pltpu.async_copy(
            msg_hbm.at[pl.ds(w * EPW + j * CHUNK, CHUNK)], buf, sem).wait()
        pltpu.sync_copy(buf, acc.at[idxv.at[j]], add=True)
        return carry

    lax.fori_loop(0, K_CH, body, 0)
    plsc.subcore_barrier()

    @pl.when(jnp.logical_and(s == 0, c == 0))
    def _out0():
        pltpu.sync_copy(acc, out0_hbm)

    @pl.when(jnp.logical_and(s == 0, c == 1))
    def _out1():
        pltpu.sync_copy(acc, out1_hbm)


def _make_sc_calls():
    mesh = plsc.VectorSubcoreMesh(core_axis_name="c", subcore_axis_name="s")
    params = pltpu.CompilerParams(use_tc_tiling_on_sc=False)
    gather = pl.kernel(
        _sc_gather_body,
        out_type=jax.ShapeDtypeStruct((E_PAD, H), jnp.float32),
        mesh=mesh,
        compiler_params=params,
        scratch_types=[
            pltpu.VMEM((K_CH, CHUNK), jnp.int32),
            pltpu.VMEM((CHUNK, H), jnp.float32),
            pltpu.SemaphoreType.DMA,
        ],
    )
    scatter = pl.kernel(
        _sc_scatter_body,
        out_type=(
            jax.ShapeDtypeStruct((N_PAD, H), jnp.float32),
            jax.ShapeDtypeStruct((N_PAD, H), jnp.float32),
        ),
        mesh=mesh,
        compiler_params=params,
        scratch_types=[
            pltpu.VMEM((K_CH, CHUNK), jnp.int32),
            pltpu.VMEM((CHUNK, H), jnp.float32),
            pltpu.VMEM_SHARED((N_PAD, H), jnp.float32),
            pltpu.SemaphoreType.DMA,
        ],
    )
    return gather, scatter


def _proj_body(nf_ref, w_ref, b_ref, out_ref):
    out_ref[...] = (
        jnp.dot(nf_ref[...], w_ref[...], preferred_element_type=jnp.float32)
        + b_ref[...])


def _edge_enc_body(efT_ref, w1t_ref, b1_ref, out_ref):
    eh = jnp.dot(w1t_ref[...], efT_ref[...],
                 preferred_element_type=jnp.float32) + b1_ref[...]
    out_ref[...] = jnp.maximum(eh, 0.0)


def _msg_body(ehT_ref, hs_ref, w2q_ref, b2q_ref, out_ref):
    hsT = hs_ref[...].T
    ehT = ehT_ref[...]
    G = (ehT[:, None, :] * hsT[None, :, :]).reshape(H * H, BE)
    msgT = lax.dot_general(
        w2q_ref[...], G, (((1,), (0,)), ((), ())),
        preferred_element_type=jnp.float32)
    msgT = msgT + jnp.dot(b2q_ref[...], hsT,
                          preferred_element_type=jnp.float32)
    out_ref[...] = msgT.T


def _gru_body(m0_ref, m1_ref, h_ref, wih_ref, whh_ref, bih_ref, bhh_ref,
              out_ref):
    h = h_ref[...]
    m = m0_ref[...] + m1_ref[...]
    gi = jnp.dot(m, wih_ref[...], preferred_element_type=jnp.float32) \
        + bih_ref[...]
    gh = jnp.dot(h, whh_ref[...], preferred_element_type=jnp.float32) \
        + bhh_ref[...]
    r = jax.nn.sigmoid(gi[:, :H] + gh[:, :H])
    z = jax.nn.sigmoid(gi[:, H:2 * H] + gh[:, H:2 * H])
    n = jnp.tanh(gi[:, 2 * H:] + r * gh[:, 2 * H:])
    out_ref[...] = (1.0 - z) * n + z * h


def _s2s_body(h_ref, seg_ref, wih_ref, whh_ref, bl_ref, wout_ref, bout_ref,
              out_ref):
    h = h_ref[...]
    seg = seg_ref[...]
    gid = lax.broadcasted_iota(jnp.int32, (1, NG), 1)
    Mt = (seg == gid).astype(jnp.float32)
    MtT = Mt.T

    q_star = jnp.zeros((NG, 2 * H), jnp.float32)
    hl = jnp.zeros((NG, H), jnp.float32)
    cl = jnp.zeros((NG, H), jnp.float32)
    for _ in range(T_S2S):
        gates = (jnp.dot(q_star, wih_ref[...],
                         preferred_element_type=jnp.float32)
                 + jnp.dot(hl, whh_ref[...],
                           preferred_element_type=jnp.float32)
                 + bl_ref[...])
        ig = jax.nn.sigmoid(gates[:, :H])
        fg = jax.nn.sigmoid(gates[:, H:2 * H])
        gg = jnp.tanh(gates[:, 2 * H:3 * H])
        og = jax.nn.sigmoid(gates[:, 3 * H:])
        cl = fg * cl + ig * gg
        hl = og * jnp.tanh(cl)
        qb = jnp.dot(Mt, hl, preferred_element_type=jnp.float32)
        e = jnp.sum(h * qb, axis=1, keepdims=True)
        S = jnp.where(Mt > 0.0, e, jnp.float32(-1e30))
        emax = jnp.max(S, axis=0, keepdims=True)
        emax_b = jnp.sum(Mt * emax, axis=1, keepdims=True)
        ex = jnp.exp(e - emax_b)
        denom = jnp.sum(Mt * ex, axis=0, keepdims=True)
        inv = 1.0 / (denom + 1e-16)
        a = ex * jnp.sum(Mt * inv, axis=1, keepdims=True)
        r_read = jnp.dot(MtT, a * h, preferred_element_type=jnp.float32)
        q_star = jnp.concatenate([hl, r_read], axis=1)
    out_ref[...] = (
        jnp.dot(q_star[:, :H], wout_ref[...],
                preferred_element_type=jnp.float32) + bout_ref[...])


@jax.jit
def _forward_impl(node_features, edge_features, Esrc, Etgt, batch,
                  W_in, b_in, ee_W1, ee_b1, ee_W2, ee_b2,
                  gru_Wih, gru_Whh, gru_bih, gru_bhh,
                  lstm_Wih, lstm_Whh, lstm_bih, lstm_bhh,
                  W_out, b_out):
    f32 = jnp.float32
    esrc = jnp.concatenate(
        [Esrc.astype(jnp.int32), jnp.zeros((E_PAD - E,), jnp.int32)]
    ).reshape(NW, K_CH, CHUNK)
    etgt = jnp.concatenate(
        [Etgt.astype(jnp.int32), jnp.full((E_PAD - E,), N, jnp.int32)]
    ).reshape(NW, K_CH, CHUNK)
    efT = jnp.pad(edge_features.astype(f32),
                  ((0, E_PAD - E), (0, 0))).T
    seg = batch.astype(jnp.int32).reshape(N, 1)
    w1t = ee_W1.T
    b1c = ee_b1.reshape(H, 1)
    w2q = ee_W2.reshape(H, H, H).transpose(1, 0, 2).reshape(H, H * H)
    b2q = ee_b2.reshape(H, H)
    zeros_n = jnp.zeros((N_PAD, H), f32)
    bl = (lstm_bih + lstm_bhh).reshape(1, 4 * H)

    gather_call, scatter_call = _make_sc_calls()

    h = pl.pallas_call(
        _proj_body,
        grid=(N // BN,),
        in_specs=[
            pl.BlockSpec((BN, DF), lambda i: (i, 0)),
            pl.BlockSpec((DF, H), lambda i: (0, 0)),
            pl.BlockSpec((1, H), lambda i: (0, 0)),
        ],
        out_specs=pl.BlockSpec((BN, H), lambda i: (i, 0)),
        out_shape=jax.ShapeDtypeStruct((N, H), f32),
    )(node_features.astype(f32), W_in, b_in.reshape(1, H))

    BEE = 4096
    ehT = pl.pallas_call(
        _edge_enc_body,
        grid=(E_PAD // BEE,),
        in_specs=[
            pl.BlockSpec((DE, BEE), lambda i: (0, i)),
            pl.BlockSpec((H, DE), lambda i: (0, 0)),
            pl.BlockSpec((H, 1), lambda i: (0, 0)),
        ],
        out_specs=pl.BlockSpec((H, BEE), lambda i: (0, i)),
        out_shape=jax.ShapeDtypeStruct((H, E_PAD), f32),
    )(efT, w1t, b1c)

    msg_call = pl.pallas_call(
        _msg_body,
        grid=(E_PAD // BE,),
        in_specs=[
            pl.BlockSpec((H, BE), lambda i: (0, i)),
            pl.BlockSpec((BE, H), lambda i: (i, 0)),
            pl.BlockSpec((H, H * H), lambda i: (0, 0)),
            pl.BlockSpec((H, H), lambda i: (0, 0)),
        ],
        out_specs=pl.BlockSpec((BE, H), lambda i: (i, 0)),
        out_shape=jax.ShapeDtypeStruct((E_PAD, H), f32),
    )
    gru_call = pl.pallas_call(
        _gru_body,
        grid=(N // BN,),
        in_specs=[
            pl.BlockSpec((BN, H), lambda i: (i, 0)),
            pl.BlockSpec((BN, H), lambda i: (i, 0)),
            pl.BlockSpec((BN, H), lambda i: (i, 0)),
            pl.BlockSpec((H, 3 * H), lambda i: (0, 0)),
            pl.BlockSpec((H, 3 * H), lambda i: (0, 0)),
            pl.BlockSpec((1, 3 * H), lambda i: (0, 0)),
            pl.BlockSpec((1, 3 * H), lambda i: (0, 0)),
        ],
        out_specs=pl.BlockSpec((BN, H), lambda i: (i, 0)),
        out_shape=jax.ShapeDtypeStruct((N, H), f32),
    )
    bih = gru_bih.reshape(1, 3 * H)
    bhh = gru_bhh.reshape(1, 3 * H)

    for _ in range(T_MP):
        hs = gather_call(h, esrc)
        msg = msg_call(ehT, hs, w2q, b2q)
        m0, m1 = scatter_call(msg, etgt, zeros_n)
        h = gru_call(m0[:N], m1[:N], h, gru_Wih, gru_Whh, bih, bhh)

    out = pl.pallas_call(
        _s2s_body,
        out_shape=jax.ShapeDtypeStruct((NG, OUTD), f32),
    )(h, seg, lstm_Wih, lstm_Whh, bl, W_out, b_out.reshape(1, OUTD))
    return out


def kernel(node_features, edge_features, Esrc, Etgt, batch,
           W_in, b_in, ee_W1, ee_b1, ee_W2, ee_b2,
           gru_Wih, gru_Whh, gru_bih, gru_bhh,
           lstm_Wih, lstm_Whh, lstm_bih, lstm_bhh,
           W_out, b_out):
    return _forward_impl(node_features, edge_features, Esrc, Etgt, batch,
                         W_in, b_in, ee_W1, ee_b1, ee_W2, ee_b2,
                         gru_Wih, gru_Whh, gru_bih, gru_bhh,
                         lstm_Wih, lstm_Whh, lstm_bih, lstm_bhh,
                         W_out, b_out)


# speedup vs baseline: 4.6712x; 1.0950x over previous
import functools

import jax
import jax.numpy as jnp
from jax import lax
from jax.experimental import pallas as pl
from jax.experimental.pallas import tpu as pltpu
from jax.experimental.pallas import tpu_sc as plsc

N = 10000
E = 160000
DF = 128
DE = 16
H = 32
OUTD = 1
NG = 16
T_MP = 3
T_S2S = 12

NC = 2
NS = 16
NW = NC * NS
CHUNK = 128
K_CH = 40
EPW = K_CH * CHUNK
E_PAD = NW * EPW
N_PAD = N + 8

BE = 1024
BN = 1000


NGRP = 4
GC = K_CH // NGRP
GROWS = GC * CHUNK


def _sc_gather_body(h_hbm, idx_hbm, out_hbm, idxv, buf0, buf1,
                    sem, wsem0, wsem1):
    c = lax.axis_index("c")
    s = lax.axis_index("s")
    w = c * NS + s
    pltpu.sync_copy(idx_hbm.at[w], idxv)

    bufs = (buf0, buf1)
    wsems = (wsem0, wsem1)
    pending = [None, None]
    for g in range(NGRP):
        b = g % 2
        if pending[b] is not None:
            pending[b].wait()
        descs = []
        for t in range(GC):
            descs.append(pltpu.async_copy(
                h_hbm.at[idxv.at[g * GC + t]],
                bufs[b].at[pl.ds(t * CHUNK, CHUNK)], sem))
        for d in descs:
            d.wait()
        pending[b] = pltpu.async_copy(
            bufs[b], out_hbm.at[pl.ds(w * EPW + g * GROWS, GROWS)],
            wsems[b])
    pending[0].wait()
    pending[1].wait()


def _sc_scatter_body(msg_hbm, idx_hbm, zeros_hbm, out0_hbm, out1_hbm,
                     idxv, buf0, buf1, acc, sem, rsem0, rsem1):
    c = lax.axis_index("c")
    s = lax.axis_index("s")
    w = c * NS + s

    @pl.when(s == 0)
    def _zero():
        pltpu.sync_copy(zeros_hbm, acc)

    pltpu.sync_copy(idx_hbm.at[w], idxv)
    plsc.subcore_barrier()

    bufs = (buf0, buf1)
    rsems = (rsem0, rsem1)
    rd = [None, None]
    rd[0] = pltpu.async_copy(
        msg_hbm.at[pl.ds(w * EPW, GROWS)], bufs[0], rsems[0])
    for g in range(NGRP):
        b = g % 2
        rd[b].wait()
        if g + 1 < NGRP:
            rd[1 - b] = pltpu.async_copy(
                msg_hbm.at[pl.ds(w * EPW + (g + 1) * GROWS, GROWS)],
                bufs[1 - b], rsems[1 - b])
        descs = []
        for t in range(GC):
            descs.append(pltpu.async_copy(
                bufs[b].at[pl.ds(t * CHUNK, CHUNK)],
                acc.at[idxv.at[g * GC + t]], sem, add=True))
        for d in descs:
            d.wait()
    plsc.subcore_barrier()

    @pl.when(jnp.logical_and(s == 0, c == 0))
    def _out0():
        pltpu.sync_copy(acc, out0_hbm)

    @pl.when(jnp.logical_and(s == 0, c == 1))
    def _out1():
        pltpu.sync_copy(acc, out1_hbm)


def _make_sc_calls():
    mesh = plsc.VectorSubcoreMesh(core_axis_name="c", subcore_axis_name="s")
    params = pltpu.CompilerParams(use_tc_tiling_on_sc=False)
    gather = pl.kernel(
        _sc_gather_body,
        out_type=jax.ShapeDtypeStruct((E_PAD, H), jnp.float32),
        mesh=mesh,
        compiler_params=params,
        scratch_types=[
            pltpu.VMEM((K_CH, CHUNK), jnp.int32),
            pltpu.VMEM((GROWS, H), jnp.float32),
            pltpu.VMEM((GROWS, H), jnp.float32),
            pltpu.SemaphoreType.DMA,
            pltpu.SemaphoreType.DMA,
            pltpu.SemaphoreType.DMA,
        ],
    )
    scatter = pl.kernel(
        _sc_scatter_body,
        out_type=(
            jax.ShapeDtypeStruct((N_PAD, H), jnp.float32),
            jax.ShapeDtypeStruct((N_PAD, H), jnp.float32),
        ),
        mesh=mesh,
        compiler_params=params,
        scratch_types=[
            pltpu.VMEM((K_CH, CHUNK), jnp.int32),
            pltpu.VMEM((GROWS, H), jnp.float32),
            pltpu.VMEM((GROWS, H), jnp.float32),
            pltpu.VMEM_SHARED((N_PAD, H), jnp.float32),
            pltpu.SemaphoreType.DMA,
            pltpu.SemaphoreType.DMA,
            pltpu.SemaphoreType.DMA,
        ],
    )
    return gather, scatter


def _proj_body(nf_ref, w_ref, b_ref, out_ref):
    out_ref[...] = (
        jnp.dot(nf_ref[...], w_ref[...], preferred_element_type=jnp.float32)
        + b_ref[...])


def _edge_enc_body(efT_ref, w1t_ref, b1_ref, out_ref):
    eh = jnp.dot(w1t_ref[...], efT_ref[...],
                 preferred_element_type=jnp.float32) + b1_ref[...]
    out_ref[...] = jnp.maximum(eh, 0.0)


def _msg_body(ehT_ref, hs_ref, w2q_ref, b2q_ref, out_ref):
    hsT = hs_ref[...].T
    ehT = ehT_ref[...]
    G = (ehT[:, None, :] * hsT[None, :, :]).reshape(H * H, BE)
    msgT = lax.dot_general(
        w2q_ref[...], G, (((1,), (0,)), ((), ())),
        preferred_element_type=jnp.float32)
    msgT = msgT + jnp.dot(b2q_ref[...], hsT,
                          preferred_element_type=jnp.float32)
    out_ref[...] = msgT.T


def _gru_body(m0_ref, m1_ref, h_ref, wih_ref, whh_ref, bih_ref, bhh_ref,
              out_ref):
    h = h_ref[...]
    m = m0_ref[...] + m1_ref[...]
    gi = jnp.dot(m, wih_ref[...], preferred_element_type=jnp.float32) \
        + bih_ref[...]
    gh = jnp.dot(h, whh_ref[...], preferred_element_type=jnp.float32) \
        + bhh_ref[...]
    r = jax.nn.sigmoid(gi[:, :H] + gh[:, :H])
    z = jax.nn.sigmoid(gi[:, H:2 * H] + gh[:, H:2 * H])
    n = jnp.tanh(gi[:, 2 * H:] + r * gh[:, 2 * H:])
    out_ref[...] = (1.0 - z) * n + z * h


def _s2s_body(h_ref, seg_ref, wih_ref, whh_ref, bl_ref, wout_ref, bout_ref,
              out_ref):
    h = h_ref[...]
    seg = seg_ref[...]
    gid = lax.broadcasted_iota(jnp.int32, (1, NG), 1)
    Mt = (seg == gid).astype(jnp.float32)
    MtT = Mt.T

    q_star = jnp.zeros((NG, 2 * H), jnp.float32)
    hl = jnp.zeros((NG, H), jnp.float32)
    cl = jnp.zeros((NG, H), jnp.float32)
    for _ in range(T_S2S):
        gates = (jnp.dot(q_star, wih_ref[...],
                         preferred_element_type=jnp.float32)
                 + jnp.dot(hl, whh_ref[...],
                           preferred_element_type=jnp.float32)
                 + bl_ref[...])
        ig = jax.nn.sigmoid(gates[:, :H])
        fg = jax.nn.sigmoid(gates[:, H:2 * H])
        gg = jnp.tanh(gates[:, 2 * H:3 * H])
        og = jax.nn.sigmoid(gates[:, 3 * H:])
        cl = fg * cl + ig * gg
        hl = og * jnp.tanh(cl)
        qb = jnp.dot(Mt, hl, preferred_element_type=jnp.float32)
        e = jnp.sum(h * qb, axis=1, keepdims=True)
        S = jnp.where(Mt > 0.0, e, jnp.float32(-1e30))
        emax = jnp.max(S, axis=0, keepdims=True)
        emax_b = jnp.sum(Mt * emax, axis=1, keepdims=True)
        ex = jnp.exp(e - emax_b)
        denom = jnp.sum(Mt * ex, axis=0, keepdims=True)
        inv = 1.0 / (denom + 1e-16)
        a = ex * jnp.sum(Mt * inv, axis=1, keepdims=True)
        r_read = jnp.dot(MtT, a * h, preferred_element_type=jnp.float32)
        q_star = jnp.concatenate([hl, r_read], axis=1)
    out_ref[...] = (
        jnp.dot(q_star[:, :H], wout_ref[...],
                preferred_element_type=jnp.float32) + bout_ref[...])


@jax.jit
def _forward_impl(node_features, edge_features, Esrc, Etgt, batch,
                  W_in, b_in, ee_W1, ee_b1, ee_W2, ee_b2,
                  gru_Wih, gru_Whh, gru_bih, gru_bhh,
                  lstm_Wih, lstm_Whh, lstm_bih, lstm_bhh,
                  W_out, b_out):
    f32 = jnp.float32
    esrc = jnp.concatenate(
        [Esrc.astype(jnp.int32), jnp.zeros((E_PAD - E,), jnp.int32)]
    ).reshape(NW, K_CH, CHUNK)
    etgt = jnp.concatenate(
        [Etgt.astype(jnp.int32), jnp.full((E_PAD - E,), N, jnp.int32)]
    ).reshape(NW, K_CH, CHUNK)
    efT = jnp.pad(edge_features.astype(f32),
                  ((0, E_PAD - E), (0, 0))).T
    seg = batch.astype(jnp.int32).reshape(N, 1)
    w1t = ee_W1.T
    b1c = ee_b1.reshape(H, 1)
    w2q = ee_W2.reshape(H, H, H).transpose(1, 0, 2).reshape(H, H * H)
    b2q = ee_b2.reshape(H, H)
    zeros_n = jnp.zeros((N_PAD, H), f32)
    bl = (lstm_bih + lstm_bhh).reshape(1, 4 * H)

    gather_call, scatter_call = _make_sc_calls()

    h = pl.pallas_call(
        _proj_body,
        grid=(N // BN,),
        in_specs=[
            pl.BlockSpec((BN, DF), lambda i: (i, 0)),
            pl.BlockSpec((DF, H), lambda i: (0, 0)),
            pl.BlockSpec((1, H), lambda i: (0, 0)),
        ],
        out_specs=pl.BlockSpec((BN, H), lambda i: (i, 0)),
        out_shape=jax.ShapeDtypeStruct((N, H), f32),
    )(node_features.astype(f32), W_in, b_in.reshape(1, H))

    BEE = 4096
    ehT = pl.pallas_call(
        _edge_enc_body,
        grid=(E_PAD // BEE,),
        in_specs=[
            pl.BlockSpec((DE, BEE), lambda i: (0, i)),
            pl.BlockSpec((H, DE), lambda i: (0, 0)),
            pl.BlockSpec((H, 1), lambda i: (0, 0)),
        ],
        out_specs=pl.BlockSpec((H, BEE), lambda i: (0, i)),
        out_shape=jax.ShapeDtypeStruct((H, E_PAD), f32),
    )(efT, w1t, b1c)

    msg_call = pl.pallas_call(
        _msg_body,
        grid=(E_PAD // BE,),
        in_specs=[
            pl.BlockSpec((H, BE), lambda i: (0, i)),
            pl.BlockSpec((BE, H), lambda i: (i, 0)),
            pl.BlockSpec((H, H * H), lambda i: (0, 0)),
            pl.BlockSpec((H, H), lambda i: (0, 0)),
        ],
        out_specs=pl.BlockSpec((BE, H), lambda i: (i, 0)),
        out_shape=jax.ShapeDtypeStruct((E_PAD, H), f32),
    )
    gru_call = pl.pallas_call(
        _gru_body,
        grid=(N // BN,),
        in_specs=[
            pl.BlockSpec((BN, H), lambda i: (i, 0)),
            pl.BlockSpec((BN, H), lambda i: (i, 0)),
            pl.BlockSpec((BN, H), lambda i: (i, 0)),
            pl.BlockSpec((H, 3 * H), lambda i: (0, 0)),
            pl.BlockSpec((H, 3 * H), lambda i: (0, 0)),
            pl.BlockSpec((1, 3 * H), lambda i: (0, 0)),
            pl.BlockSpec((1, 3 * H), lambda i: (0, 0)),
        ],
        out_specs=pl.BlockSpec((BN, H), lambda i: (i, 0)),
        out_shape=jax.ShapeDtypeStruct((N, H), f32),
    )
    bih = gru_bih.reshape(1, 3 * H)
    bhh = gru_bhh.reshape(1, 3 * H)

    for _ in range(T_MP):
        hs = gather_call(h, esrc)
        msg = msg_call(ehT, hs, w2q, b2q)
        m0, m1 = scatter_call(msg, etgt, zeros_n)
        h = gru_call(m0[:N], m1[:N], h, gru_Wih, gru_Whh, bih, bhh)

    out = pl.pallas_call(
        _s2s_body,
        out_shape=jax.ShapeDtypeStruct((NG, OUTD), f32),
    )(h, seg, lstm_Wih, lstm_Whh, bl, W_out, b_out.reshape(1, OUTD))
    return out


def kernel(node_features, edge_features, Esrc, Etgt, batch,
           W_in, b_in, ee_W1, ee_b1, ee_W2, ee_b2,
           gru_Wih, gru_Whh, gru_bih, gru_bhh,
           lstm_Wih, lstm_Whh, lstm_bih, lstm_bhh,
           W_out, b_out):
    return _forward_impl(node_features, edge_features, Esrc, Etgt, batch,
                         W_in, b_in, ee_W1, ee_b1, ee_W2, ee_b2,
                         gru_Wih, gru_Whh, gru_bih, gru_bhh,
                         lstm_Wih, lstm_Whh, lstm_bih, lstm_bhh,
                         W_out, b_out)


# speedup vs baseline: 7.4026x; 1.5847x over previous
import functools

import jax
import jax.numpy as jnp
from jax import lax
from jax.experimental import pallas as pl
from jax.experimental.pallas import tpu as pltpu
from jax.experimental.pallas import tpu_sc as plsc

N = 10000
E = 160000
DF = 128
DE = 16
H = 32
OUTD = 1
NG = 16
T_MP = 3
T_S2S = 12

NC = 2
NS = 16
NW = NC * NS
CHUNK = 128
K_CH = 40
EPW = K_CH * CHUNK
E_PAD = NW * EPW
N_PAD = N + 8

RB = 512
BE = 4 * RB
NB = E_PAD // BE
BN = 1000


NGRP = 4
GC = K_CH // NGRP
GROWS = GC * CHUNK


def _sc_gather_body(h_hbm, idx_hbm, out_hbm, idxv, buf0, buf1,
                    sem, wsem0, wsem1):
    c = lax.axis_index("c")
    s = lax.axis_index("s")
    w = c * NS + s
    pltpu.sync_copy(idx_hbm.at[w], idxv)

    bufs = (buf0, buf1)
    wsems = (wsem0, wsem1)
    pending = [None, None]
    for g in range(NGRP):
        b = g % 2
        if pending[b] is not None:
            pending[b].wait()
        descs = []
        for t in range(GC):
            descs.append(pltpu.async_copy(
                h_hbm.at[idxv.at[g * GC + t]],
                bufs[b].at[pl.ds(t * CHUNK, CHUNK)], sem))
        for d in descs:
            d.wait()
        pending[b] = pltpu.async_copy(
            bufs[b], out_hbm.at[pl.ds(w * EPW + g * GROWS, GROWS)],
            wsems[b])
    pending[0].wait()
    pending[1].wait()


def _sc_scatter_body(msg_hbm, idx_hbm, zeros_hbm, out0_hbm, out1_hbm,
                     idxv, buf0, buf1, acc, sem, rsem0, rsem1):
    c = lax.axis_index("c")
    s = lax.axis_index("s")
    w = c * NS + s

    @pl.when(s == 0)
    def _zero():
        pltpu.sync_copy(zeros_hbm, acc)

    pltpu.sync_copy(idx_hbm.at[w], idxv)
    plsc.subcore_barrier()

    bufs = (buf0, buf1)
    rsems = (rsem0, rsem1)
    rd = [None, None]
    rd[0] = pltpu.async_copy(
        msg_hbm.at[pl.ds(w * EPW, GROWS)], bufs[0], rsems[0])
    for g in range(NGRP):
        b = g % 2
        rd[b].wait()
        if g + 1 < NGRP:
            rd[1 - b] = pltpu.async_copy(
                msg_hbm.at[pl.ds(w * EPW + (g + 1) * GROWS, GROWS)],
                bufs[1 - b], rsems[1 - b])
        descs = []
        for t in range(GC):
            descs.append(pltpu.async_copy(
                bufs[b].at[pl.ds(t * CHUNK, CHUNK)],
                acc.at[idxv.at[g * GC + t]], sem, add=True))
        for d in descs:
            d.wait()
    plsc.subcore_barrier()

    @pl.when(jnp.logical_and(s == 0, c == 0))
    def _out0():
        pltpu.sync_copy(acc, out0_hbm)

    @pl.when(jnp.logical_and(s == 0, c == 1))
    def _out1():
        pltpu.sync_copy(acc, out1_hbm)


def _make_sc_calls():
    mesh = plsc.VectorSubcoreMesh(core_axis_name="c", subcore_axis_name="s")
    params = pltpu.CompilerParams(use_tc_tiling_on_sc=False)
    gather = pl.kernel(
        _sc_gather_body,
        out_type=jax.ShapeDtypeStruct((E_PAD, H), jnp.float32),
        mesh=mesh,
        compiler_params=params,
        scratch_types=[
            pltpu.VMEM((K_CH, CHUNK), jnp.int32),
            pltpu.VMEM((GROWS, H), jnp.float32),
            pltpu.VMEM((GROWS, H), jnp.float32),
            pltpu.SemaphoreType.DMA,
            pltpu.SemaphoreType.DMA,
            pltpu.SemaphoreType.DMA,
        ],
    )
    scatter = pl.kernel(
        _sc_scatter_body,
        out_type=(
            jax.ShapeDtypeStruct((N_PAD, H), jnp.float32),
            jax.ShapeDtypeStruct((N_PAD, H), jnp.float32),
        ),
        mesh=mesh,
        compiler_params=params,
        scratch_types=[
            pltpu.VMEM((K_CH, CHUNK), jnp.int32),
            pltpu.VMEM((GROWS, H), jnp.float32),
            pltpu.VMEM((GROWS, H), jnp.float32),
            pltpu.VMEM_SHARED((N_PAD, H), jnp.float32),
            pltpu.SemaphoreType.DMA,
            pltpu.SemaphoreType.DMA,
            pltpu.SemaphoreType.DMA,
        ],
    )
    return gather, scatter


def _proj_body(nf_ref, w_ref, b_ref, out_ref):
    out_ref[...] = (
        jnp.dot(nf_ref[...], w_ref[...], preferred_element_type=jnp.float32)
        + b_ref[...])


def _edge_enc_body(efT_ref, w1t_ref, b1_ref, out_ref):
    eh = jnp.dot(w1t_ref[...], efT_ref[...],
                 preferred_element_type=jnp.float32) + b1_ref[...]
    out_ref[...] = jnp.maximum(eh, 0.0)


def _msg_body(ehT_ref, hsp_ref, w2q_ref, b2q_ref, out_ref):
    XT = hsp_ref[...].T
    ehT = ehT_ref[...]
    parts = []
    for p in range(4):
        hsT = XT[p * H:(p + 1) * H, :]
        ehTp = ehT[:, p * RB:(p + 1) * RB]
        G = (ehTp[:, None, :] * hsT[None, :, :]).reshape(H * H, RB)
        msgT = lax.dot_general(
            w2q_ref[...], G, (((1,), (0,)), ((), ())),
            preferred_element_type=jnp.float32)
        msgT = msgT + jnp.dot(b2q_ref[...], hsT,
                              preferred_element_type=jnp.float32)
        parts.append(msgT)
    out_ref[...] = jnp.concatenate(parts, axis=0).T


def _gru_body(m0_ref, m1_ref, h_ref, wih_ref, whh_ref, bih_ref, bhh_ref,
              out_ref):
    h = h_ref[...]
    m = m0_ref[...] + m1_ref[...]
    gi = jnp.dot(m, wih_ref[...], preferred_element_type=jnp.float32) \
        + bih_ref[...]
    gh = jnp.dot(h, whh_ref[...], preferred_element_type=jnp.float32) \
        + bhh_ref[...]
    r = jax.nn.sigmoid(gi[:, :H] + gh[:, :H])
    z = jax.nn.sigmoid(gi[:, H:2 * H] + gh[:, H:2 * H])
    n = jnp.tanh(gi[:, 2 * H:] + r * gh[:, 2 * H:])
    out_ref[...] = (1.0 - z) * n + z * h


def _s2s_body(h_ref, seg_ref, wih_ref, whh_ref, bl_ref, wout_ref, bout_ref,
              out_ref):
    h = h_ref[...]
    seg = seg_ref[...]
    gid = lax.broadcasted_iota(jnp.int32, (1, NG), 1)
    Mt = (seg == gid).astype(jnp.float32)
    MtT = Mt.T

    q_star = jnp.zeros((NG, 2 * H), jnp.float32)
    hl = jnp.zeros((NG, H), jnp.float32)
    cl = jnp.zeros((NG, H), jnp.float32)
    for _ in range(T_S2S):
        gates = (jnp.dot(q_star, wih_ref[...],
                         preferred_element_type=jnp.float32)
                 + jnp.dot(hl, whh_ref[...],
                           preferred_element_type=jnp.float32)
                 + bl_ref[...])
        ig = jax.nn.sigmoid(gates[:, :H])
        fg = jax.nn.sigmoid(gates[:, H:2 * H])
        gg = jnp.tanh(gates[:, 2 * H:3 * H])
        og = jax.nn.sigmoid(gates[:, 3 * H:])
        cl = fg * cl + ig * gg
        hl = og * jnp.tanh(cl)
        qb = jnp.dot(Mt, hl, preferred_element_type=jnp.float32)
        e = jnp.sum(h * qb, axis=1, keepdims=True)
        S = jnp.where(Mt > 0.0, e, jnp.float32(-1e30))
        emax = jnp.max(S, axis=0, keepdims=True)
        emax_b = jnp.sum(Mt * emax, axis=1, keepdims=True)
        ex = jnp.exp(e - emax_b)
        denom = jnp.sum(Mt * ex, axis=0, keepdims=True)
        inv = 1.0 / (denom + 1e-16)
        a = ex * jnp.sum(Mt * inv, axis=1, keepdims=True)
        r_read = jnp.dot(MtT, a * h, preferred_element_type=jnp.float32)
        q_star = jnp.concatenate([hl, r_read], axis=1)
    out_ref[...] = (
        jnp.dot(q_star[:, :H], wout_ref[...],
                preferred_element_type=jnp.float32) + bout_ref[...])


@jax.jit
def _forward_impl(node_features, edge_features, Esrc, Etgt, batch,
                  W_in, b_in, ee_W1, ee_b1, ee_W2, ee_b2,
                  gru_Wih, gru_Whh, gru_bih, gru_bhh,
                  lstm_Wih, lstm_Whh, lstm_bih, lstm_bhh,
                  W_out, b_out):
    f32 = jnp.float32
    esrc = jnp.concatenate(
        [Esrc.astype(jnp.int32), jnp.zeros((E_PAD - E,), jnp.int32)]
    ).reshape(NW, K_CH, CHUNK)
    etgt = jnp.concatenate(
        [Etgt.astype(jnp.int32), jnp.full((E_PAD - E,), N, jnp.int32)]
    ).reshape(NW, K_CH, CHUNK)
    efT = jnp.pad(edge_features.astype(f32),
                  ((0, E_PAD - E), (0, 0))) \
        .reshape(NB, RB, 4, DE).transpose(3, 0, 2, 1) \
        .reshape(DE, E_PAD)
    seg = batch.astype(jnp.int32).reshape(N, 1)
    w1t = ee_W1.T
    b1c = ee_b1.reshape(H, 1)
    w2q = ee_W2.reshape(H, H, H).transpose(1, 0, 2).reshape(H, H * H)
    b2q = ee_b2.reshape(H, H)
    zeros_n = jnp.zeros((N_PAD, H), f32)
    bl = (lstm_bih + lstm_bhh).reshape(1, 4 * H)

    gather_call, scatter_call = _make_sc_calls()

    h = pl.pallas_call(
        _proj_body,
        grid=(N // BN,),
        in_specs=[
            pl.BlockSpec((BN, DF), lambda i: (i, 0)),
            pl.BlockSpec((DF, H), lambda i: (0, 0)),
            pl.BlockSpec((1, H), lambda i: (0, 0)),
        ],
        out_specs=pl.BlockSpec((BN, H), lambda i: (i, 0)),
        out_shape=jax.ShapeDtypeStruct((N, H), f32),
    )(node_features.astype(f32), W_in, b_in.reshape(1, H))

    BEE = 4096
    ehT = pl.pallas_call(
        _edge_enc_body,
        grid=(E_PAD // BEE,),
        in_specs=[
            pl.BlockSpec((DE, BEE), lambda i: (0, i)),
            pl.BlockSpec((H, DE), lambda i: (0, 0)),
            pl.BlockSpec((H, 1), lambda i: (0, 0)),
        ],
        out_specs=pl.BlockSpec((H, BEE), lambda i: (0, i)),
        out_shape=jax.ShapeDtypeStruct((H, E_PAD), f32),
    )(efT, w1t, b1c)

    msg_call = pl.pallas_call(
        _msg_body,
        grid=(NB,),
        in_specs=[
            pl.BlockSpec((H, BE), lambda i: (0, i)),
            pl.BlockSpec((RB, 128), lambda i: (i, 0)),
            pl.BlockSpec((H, H * H), lambda i: (0, 0)),
            pl.BlockSpec((H, H), lambda i: (0, 0)),
        ],
        out_specs=pl.BlockSpec((RB, 128), lambda i: (i, 0)),
        out_shape=jax.ShapeDtypeStruct((E_PAD // 4, 128), f32),
    )
    gru_call = pl.pallas_call(
        _gru_body,
        grid=(N // BN,),
        in_specs=[
            pl.BlockSpec((BN, H), lambda i: (i, 0)),
            pl.BlockSpec((BN, H), lambda i: (i, 0)),
            pl.BlockSpec((BN, H), lambda i: (i, 0)),
            pl.BlockSpec((H, 3 * H), lambda i: (0, 0)),
            pl.BlockSpec((H, 3 * H), lambda i: (0, 0)),
            pl.BlockSpec((1, 3 * H), lambda i: (0, 0)),
            pl.BlockSpec((1, 3 * H), lambda i: (0, 0)),
        ],
        out_specs=pl.BlockSpec((BN, H), lambda i: (i, 0)),
        out_shape=jax.ShapeDtypeStruct((N, H), f32),
    )
    bih = gru_bih.reshape(1, 3 * H)
    bhh = gru_bhh.reshape(1, 3 * H)

    for _ in range(T_MP):
        hs = gather_call(h, esrc)
        hsp = hs.reshape(E_PAD // 4, 128)
        msgp = msg_call(ehT, hsp, w2q, b2q)
        msg = msgp.reshape(E_PAD, H)
        m0, m1 = scatter_call(msg, etgt, zeros_n)
        h = gru_call(m0[:N], m1[:N], h, gru_Wih, gru_Whh, bih, bhh)

    out = pl.pallas_call(
        _s2s_body,
        out_shape=jax.ShapeDtypeStruct((NG, OUTD), f32),
    )(h, seg, lstm_Wih, lstm_Whh, bl, W_out, b_out.reshape(1, OUTD))
    return out


def kernel(node_features, edge_features, Esrc, Etgt, batch,
           W_in, b_in, ee_W1, ee_b1, ee_W2, ee_b2,
           gru_Wih, gru_Whh, gru_bih, gru_bhh,
           lstm_Wih, lstm_Whh, lstm_bih, lstm_bhh,
           W_out, b_out):
    return _forward_impl(node_features, edge_features, Esrc, Etgt, batch,
                         W_in, b_in, ee_W1, ee_b1, ee_W2, ee_b2,
                         gru_Wih, gru_Whh, gru_bih, gru_bhh,
                         lstm_Wih, lstm_Whh, lstm_bih, lstm_bhh,
                         W_out, b_out)
